# baseline (speedup 1.0000x reference)
import jax
import jax.numpy as jnp
from jax.experimental import pallas as pl
from jax.experimental.pallas import tpu as pltpu

B, N, S, D1, D2, DO = 8, 4096, 1024, 128, 256, 128
NBLK = 512
NB = N // NBLK


def _prep_body(f2_ref, x2t_ref, w1_ref, b1_ref, w2_ref, b2_ref,
               proj2_ref, w_ref, bias_ref, x2aug_ref):
    w = jnp.dot(w2_ref[...], w1_ref[...], preferred_element_type=jnp.float32)
    w_ref[...] = w
    bias_ref[...] = jnp.dot(w2_ref[...], b1_ref[...],
                            preferred_element_type=jnp.float32) + b2_ref[...]
    wb = w[:, D1:]
    proj2_ref[0] = jnp.dot(f2_ref[0], wb.T,
                           preferred_element_type=jnp.float32).astype(jnp.bfloat16)
    x2t = x2t_ref[0]
    x2aug_ref[0] = -2.0 * x2t


def _main_body(x1_ref, x2aug_ref, f1_ref, proj2_ref, w_ref, bias_ref, out_ref):
    x2t = x2aug_ref[0]
    n2 = jnp.sum(x2t * x2t, axis=0, keepdims=True) * 0.25
    d = n2 + jnp.dot(x1_ref[0], x2t, preferred_element_type=jnp.float32)
    acc = None
    for k in range(3):
        m = jnp.min(d, axis=1, keepdims=True)
        sel = d == m
        acc = sel if acc is None else acc | sel
        if k < 2:
            d = jnp.where(sel, jnp.float32(jnp.inf), d)
    onehot = acc.astype(jnp.bfloat16)
    interp_t = jax.lax.dot_general(
        proj2_ref[0], onehot, (((0,), (1,)), ((), ())),
        preferred_element_type=jnp.float32)
    wa = w_ref[:, :D1].astype(jnp.bfloat16)
    base_t = jax.lax.dot_general(
        wa, f1_ref[0], (((1,), (1,)), ((), ())),
        preferred_element_type=jnp.float32)
    out_ref[0] = base_t + interp_t * (1.0 / 3.0) + bias_ref[...]


def kernel(xyz1, xyz2, features1, features2, W1, b1, W2, b2):
    ones = jnp.ones((B, N, 1), jnp.float32)
    zeros = jnp.zeros((B, N, 4), jnp.float32)
    xyz1p = jnp.concatenate([xyz1, ones, zeros], axis=2)
    xyz2t = jnp.pad(xyz2, ((0, 0), (0, 0), (0, 5)))
    xyz2t = jnp.transpose(xyz2t, (0, 2, 1))
    f1b = features1.astype(jnp.bfloat16)
    b1r = b1.reshape(D2, 1)
    b2r = b2.reshape(DO, 1)

    proj2, w, bias, x2aug = pl.pallas_call(
        _prep_body,
        grid=(B,),
        in_specs=[
            pl.BlockSpec((1, S, D2), lambda b: (b, 0, 0)),
            pl.BlockSpec((1, 8, S), lambda b: (b, 0, 0)),
            pl.BlockSpec((D2, D1 + D2), lambda b: (0, 0)),
            pl.BlockSpec((D2, 1), lambda b: (0, 0)),
            pl.BlockSpec((DO, D2), lambda b: (0, 0)),
            pl.BlockSpec((DO, 1), lambda b: (0, 0)),
        ],
        out_specs=[
            pl.BlockSpec((1, S, DO), lambda b: (b, 0, 0)),
            pl.BlockSpec((DO, D1 + D2), lambda b: (0, 0)),
            pl.BlockSpec((DO, 1), lambda b: (0, 0)),
            pl.BlockSpec((1, 8, S), lambda b: (b, 0, 0)),
        ],
        out_shape=[
            jax.ShapeDtypeStruct((B, S, DO), jnp.bfloat16),
            jax.ShapeDtypeStruct((DO, D1 + D2), jnp.float32),
            jax.ShapeDtypeStruct((DO, 1), jnp.float32),
            jax.ShapeDtypeStruct((B, 8, S), jnp.float32),
        ],
    )(features2, xyz2t, W1, b1r, W2, b2r)

    out = pl.pallas_call(
        _main_body,
        grid=(B, NB),
        in_specs=[
            pl.BlockSpec((1, NBLK, 8), lambda b, nb: (b, nb, 0)),
            pl.BlockSpec((1, 8, S), lambda b, nb: (b, 0, 0)),
            pl.BlockSpec((1, NBLK, D1), lambda b, nb: (b, nb, 0)),
            pl.BlockSpec((1, S, DO), lambda b, nb: (b, 0, 0)),
            pl.BlockSpec((DO, D1 + D2), lambda b, nb: (0, 0)),
            pl.BlockSpec((DO, 1), lambda b, nb: (0, 0)),
        ],
        out_specs=pl.BlockSpec((1, DO, NBLK), lambda b, nb: (b, 0, nb)),
        out_shape=jax.ShapeDtypeStruct((B, DO, N), jnp.float32),
    )(xyz1p, x2aug, f1b, proj2, w, bias)
    return out


# speedup vs baseline: 1.4000x; 1.4000x over previous
import jax
import jax.numpy as jnp
from jax.experimental import pallas as pl

B, N, S, D1, D2, DO = 8, 4096, 1024, 128, 256, 128
NBLK = 1024
NB = N // NBLK


def _prep_body(f2_ref, x2t_ref, w1_ref, b1_ref, w2_ref, b2_ref,
               proj2_ref, w_ref, bias_ref, x2m2_ref, n2_ref):
    w = jnp.dot(w2_ref[...], w1_ref[...], preferred_element_type=jnp.float32)
    w_ref[...] = w
    bias_ref[...] = jnp.dot(w2_ref[...], b1_ref[...],
                            preferred_element_type=jnp.float32) + b2_ref[...]
    wb = w[:, D1:]
    proj2_ref[0] = jnp.dot(f2_ref[0], wb.T, preferred_element_type=jnp.float32)
    x2t = x2t_ref[0]
    n2_ref[0] = jnp.sum(x2t * x2t, axis=0, keepdims=True)
    x2m2_ref[0] = -2.0 * x2t


def _main_body(x1_ref, x2m2_ref, n2_ref, f1_ref, proj2_ref, w_ref, bias_ref,
               out_ref):
    d0 = n2_ref[0] + jnp.dot(x1_ref[0], x2m2_ref[0],
                             preferred_element_type=jnp.float32)
    m1 = jnp.min(d0, axis=1, keepdims=True)
    d1 = jnp.where(d0 == m1, jnp.float32(jnp.inf), d0)
    m2 = jnp.min(d1, axis=1, keepdims=True)
    d2 = jnp.where(d1 == m2, jnp.float32(jnp.inf), d1)
    m3 = jnp.min(d2, axis=1, keepdims=True)
    onehot = (d0 <= m3).astype(jnp.float32)
    interp_t = jax.lax.dot_general(
        proj2_ref[0], onehot, (((0,), (1,)), ((), ())),
        preferred_element_type=jnp.float32)
    base_t = jax.lax.dot_general(
        w_ref[:, :D1], f1_ref[0], (((1,), (1,)), ((), ())),
        preferred_element_type=jnp.float32)
    out_ref[0] = base_t + interp_t * (1.0 / 3.0) + bias_ref[...]


def kernel(xyz1, xyz2, features1, features2, W1, b1, W2, b2):
    xyz1p = jnp.pad(xyz1, ((0, 0), (0, 0), (0, 5)))
    xyz2t = jnp.pad(xyz2, ((0, 0), (0, 0), (0, 5)))
    xyz2t = jnp.transpose(xyz2t, (0, 2, 1))
    b1r = b1.reshape(D2, 1)
    b2r = b2.reshape(DO, 1)

    proj2, w, bias, x2m2, n2 = pl.pallas_call(
        _prep_body,
        grid=(B,),
        in_specs=[
            pl.BlockSpec((1, S, D2), lambda b: (b, 0, 0)),
            pl.BlockSpec((1, 8, S), lambda b: (b, 0, 0)),
            pl.BlockSpec((D2, D1 + D2), lambda b: (0, 0)),
            pl.BlockSpec((D2, 1), lambda b: (0, 0)),
            pl.BlockSpec((DO, D2), lambda b: (0, 0)),
            pl.BlockSpec((DO, 1), lambda b: (0, 0)),
        ],
        out_specs=[
            pl.BlockSpec((1, S, DO), lambda b: (b, 0, 0)),
            pl.BlockSpec((DO, D1 + D2), lambda b: (0, 0)),
            pl.BlockSpec((DO, 1), lambda b: (0, 0)),
            pl.BlockSpec((1, 8, S), lambda b: (b, 0, 0)),
            pl.BlockSpec((1, 1, S), lambda b: (b, 0, 0)),
        ],
        out_shape=[
            jax.ShapeDtypeStruct((B, S, DO), jnp.float32),
            jax.ShapeDtypeStruct((DO, D1 + D2), jnp.float32),
            jax.ShapeDtypeStruct((DO, 1), jnp.float32),
            jax.ShapeDtypeStruct((B, 8, S), jnp.float32),
            jax.ShapeDtypeStruct((B, 1, S), jnp.float32),
        ],
    )(features2, xyz2t, W1, b1r, W2, b2r)

    out = pl.pallas_call(
        _main_body,
        grid=(B, NB),
        in_specs=[
            pl.BlockSpec((1, NBLK, 8), lambda b, nb: (b, nb, 0)),
            pl.BlockSpec((1, 8, S), lambda b, nb: (b, 0, 0)),
            pl.BlockSpec((1, 1, S), lambda b, nb: (b, 0, 0)),
            pl.BlockSpec((1, NBLK, D1), lambda b, nb: (b, nb, 0)),
            pl.BlockSpec((1, S, DO), lambda b, nb: (b, 0, 0)),
            pl.BlockSpec((DO, D1 + D2), lambda b, nb: (0, 0)),
            pl.BlockSpec((DO, 1), lambda b, nb: (0, 0)),
        ],
        out_specs=pl.BlockSpec((1, DO, NBLK), lambda b, nb: (b, 0, nb)),
        out_shape=jax.ShapeDtypeStruct((B, DO, N), jnp.float32),
    )(xyz1p, x2m2, n2, features1, proj2, w, bias)
    return out
